# baseline (speedup 1.0000x reference)
import functools

import jax
import jax.numpy as jnp
from jax import lax
from jax.experimental import pallas as pl
from jax.experimental.pallas import tpu as pltpu
from jax.experimental.pallas import tpu_sc as plsc

_N, _C, _H, _W = 32, 384, 56, 56

_PERM = (
    121, 35, 130, 263, 148, 197, 45, 176, 366, 257, 179, 139, 315, 188, 312,
    318, 304, 99, 309, 144, 152, 189, 325, 31, 112, 356, 268, 85, 63, 117,
    174, 272, 114, 254, 82, 65, 7, 350, 4, 101, 102, 78, 163, 157, 302, 183,
    29, 240, 177, 278, 259, 108, 305, 83, 129, 367, 212, 277, 300, 44, 211,
    16, 58, 123, 37, 336, 111, 19, 61, 2, 142, 34, 369, 339, 156, 5, 90, 363,
    175, 167, 284, 379, 251, 110, 72, 155, 178, 323, 291, 269, 354, 368, 219,
    153, 30, 275, 42, 186, 342, 307, 256, 246, 3, 362, 380, 327, 70, 378,
    271, 311, 67, 273, 223, 39, 56, 274, 192, 169, 349, 218, 195, 173, 245,
    241, 69, 383, 80, 22, 6, 321, 199, 345, 118, 235, 54, 266, 77, 147, 18,
    340, 298, 249, 294, 375, 382, 10, 11, 234, 53, 236, 94, 332, 331, 353,
    287, 32, 217, 283, 355, 159, 15, 184, 49, 137, 50, 138, 20, 237, 280,
    253, 185, 43, 335, 258, 370, 344, 92, 8, 324, 140, 233, 24, 81, 239, 314,
    96, 154, 135, 264, 160, 106, 128, 265, 191, 9, 200, 40, 187, 71, 346,
    333, 248, 164, 207, 93, 59, 201, 158, 210, 75, 131, 97, 66, 25, 196, 364,
    242, 338, 206, 243, 341, 238, 295, 308, 73, 320, 13, 52, 203, 289, 303,
    202, 255, 194, 88, 250, 337, 62, 230, 150, 261, 330, 262, 209, 132, 357,
    87, 76, 198, 60, 244, 47, 374, 276, 33, 79, 180, 247, 14, 286, 228, 17,
    38, 86, 231, 190, 232, 23, 105, 301, 376, 313, 220, 0, 145, 371, 213,
    226, 381, 133, 281, 41, 64, 21, 161, 279, 285, 166, 124, 116, 26, 165,
    168, 193, 57, 208, 181, 89, 146, 182, 126, 125, 297, 1, 115, 28, 113,
    225, 361, 351, 172, 377, 162, 48, 170, 227, 36, 252, 119, 151, 306, 120,
    372, 224, 122, 270, 100, 329, 365, 91, 222, 55, 103, 51, 293, 215, 127,
    98, 282, 107, 27, 322, 74, 136, 229, 319, 328, 343, 204, 221, 296, 12,
    134, 109, 84, 317, 358, 299, 205, 171, 288, 143, 68, 267, 216, 149, 141,
    334, 104, 352, 95, 316, 214, 290, 46, 310, 348, 260, 292, 359, 326, 347,
    373, 360,
)

_NC, _NS = 2, 16
_NW = _NC * _NS
_CPW = _C // _NW
_L = 16

import numpy as np
_SRC_TABLE = np.zeros((_NW, _L), dtype=np.int32)
for _w in range(_NW):
    _SRC_TABLE[_w, :_CPW] = _PERM[_w * _CPW:(_w + 1) * _CPW]

_mesh = plsc.VectorSubcoreMesh(core_axis_name="c", subcore_axis_name="s")


@functools.partial(
    pl.kernel,
    mesh=_mesh,
    out_type=jax.ShapeDtypeStruct((_N, _C, _H, _W), jnp.float32),
    scratch_types=[
        pltpu.VMEM((_L,), jnp.int32),
        pltpu.SemaphoreType.DMA,
    ],
)
def _shuffle(x_hbm, src_hbm, out_hbm, idx_v, sem):
    wid = lax.axis_index("s") * _NC + lax.axis_index("c")
    pltpu.sync_copy(src_hbm.at[wid], idx_v)
    vec = idx_v[...]
    c0 = wid * _CPW
    copies = []
    for j in range(_CPW):
        pc = vec[j]
        copies.append(
            pltpu.make_async_copy(x_hbm.at[:, pc], out_hbm.at[:, c0 + j], sem))
        copies[-1].start()
    for cp in copies:
        cp.wait()


def kernel(input):
    return _shuffle(input, jnp.asarray(_SRC_TABLE))


# speedup vs baseline: 15.1609x; 15.1609x over previous
import functools

import numpy as np
import jax
import jax.numpy as jnp
from jax import lax
from jax.experimental import pallas as pl
from jax.experimental.pallas import tpu as pltpu
from jax.experimental.pallas import tpu_sc as plsc

_N, _C, _H, _W = 32, 384, 56, 56
_B = _N * _C


def _perm_indices():
    return jax.random.permutation(jax.random.key(42), _C).astype(jnp.int32)


_NC, _NS = 2, 16
_NW = _NC * _NS
_CH = 8
_NCHUNK = _C // _CH

_mesh = plsc.VectorSubcoreMesh(core_axis_name="c", subcore_axis_name="s")


@functools.partial(
    pl.kernel,
    mesh=_mesh,
    out_type=jax.ShapeDtypeStruct((_N, _C, _H, _W), jnp.float32),
    scratch_types=[
        pltpu.VMEM((_C,), jnp.int32),
        pltpu.VMEM((_CH, _H, _W), jnp.float32),
        pltpu.VMEM((_CH, _H, _W), jnp.float32),
        pltpu.SemaphoreType.DMA,
        pltpu.SemaphoreType.DMA,
        pltpu.SemaphoreType.DMA,
        pltpu.SemaphoreType.DMA,
    ],
)
def _shuffle_rows(x_hbm, idx_hbm, out_hbm, idx_v, buf0, buf1, g0, g1, s0, s1):
    bufs, gsems, ssems = (buf0, buf1), (g0, g1), (s0, s1)
    wid = lax.axis_index("s") * _NC + lax.axis_index("c")
    pltpu.sync_copy(idx_hbm, idx_v)

    def g_start(srcs, b):
        for j in range(_CH):
            pltpu.make_async_copy(
                x_hbm.at[wid, srcs[j]], bufs[b].at[j], gsems[b]).start()

    def g_wait(b):
        for j in range(_CH):
            pltpu.make_async_copy(
                x_hbm.at[wid, 0], bufs[b].at[j], gsems[b]).wait()

    def s_copy(v, b):
        return pltpu.make_async_copy(
            bufs[b], out_hbm.at[wid, pl.ds(v * _CH, _CH)], ssems[b])

    vec0 = idx_v[pl.ds(0, 16)]
    g_start([vec0[j] for j in range(_CH)], 0)

    @pl.loop(0, _NCHUNK, step=2)
    def _(i):
        vec = idx_v[pl.ds(i * _CH, 16)]
        for b in range(2):
            v = i + b
            g_wait(b)
            s_copy(v, b).start()

            @pl.when(v + 1 < _NCHUNK)
            def _start_next():
                @pl.when(v >= 1)
                def _drain_prev():
                    s_copy(v - 1, 1 - b).wait()
                if b == 0:
                    g_start([vec[_CH + j] for j in range(_CH)], 1)
                else:
                    vec2 = idx_v[pl.ds((i + 2) * _CH, 16)]
                    g_start([vec2[j] for j in range(_CH)], 0)

    s_copy(_NCHUNK - 2, 0).wait()
    s_copy(_NCHUNK - 1, 1).wait()


def kernel(input):
    return _shuffle_rows(input, _perm_indices())


# speedup vs baseline: 29.8439x; 1.9685x over previous
import functools

import jax
import jax.numpy as jnp
from jax import lax
from jax.experimental import pallas as pl
from jax.experimental.pallas import tpu as pltpu
from jax.experimental.pallas import tpu_sc as plsc

_N, _C, _H, _W = 32, 384, 56, 56
_L = 16
_CT = _C // 128
_GROUPS = _N * _H * (_W // 8)
_GROW = _CT * 8 * 128


def _gather_base():
    perm = jax.random.permutation(jax.random.key(42), _C).astype(jnp.int32)
    return (perm // 128) * 1024 + perm % 128


_NC, _NS = 2, 16
_NW = _NC * _NS
_GPW = _GROUPS // _NW
_GCH = 8
_NCHUNK = _GPW // _GCH
_CHW = _GCH * _GROW
_KB = _C // _L

_mesh = plsc.VectorSubcoreMesh(core_axis_name="c", subcore_axis_name="s")


@functools.partial(
    pl.kernel,
    mesh=_mesh,
    out_type=jax.ShapeDtypeStruct((_GROUPS, _GROW), jnp.float32),
    compiler_params=pltpu.CompilerParams(
        use_tc_tiling_on_sc=False, needs_layout_passes=False),
    scratch_types=[
        pltpu.VMEM((_C,), jnp.int32),
        pltpu.VMEM((_GCH, _GROW), jnp.float32),
        pltpu.VMEM((_GCH, _GROW), jnp.float32),
        pltpu.VMEM((_GCH, _GROW), jnp.float32),
        pltpu.VMEM((_GCH, _GROW), jnp.float32),
        pltpu.SemaphoreType.DMA,
        pltpu.SemaphoreType.DMA,
        pltpu.SemaphoreType.DMA,
        pltpu.SemaphoreType.DMA,
    ],
)
def _shuffle_groups(x_hbm, idx_hbm, out_hbm,
                    idx_v, in0, in1, ou0, ou1, g0, g1, s0, s1):
    ins, ous, gsems, ssems = (in0, in1), (ou0, ou1), (g0, g1), (s0, s1)
    wid = lax.axis_index("s") * _NC + lax.axis_index("c")
    base = wid * _GPW
    pltpu.sync_copy(idx_hbm, idx_v)

    def g_copy(v, b):
        return pltpu.make_async_copy(
            x_hbm.at[pl.ds(base + v * _GCH, _GCH)], ins[b], gsems[b])

    def s_copy(v, b):
        return pltpu.make_async_copy(
            ous[b], out_hbm.at[pl.ds(base + v * _GCH, _GCH)], ssems[b])

    def permute(b):
        for k in range(_KB):
            iv = idx_v[pl.ds(k * _L, _L)]
            ct, m = divmod(k, 8)
            dst0 = ct * 1024 + m * _L

            def body(q, carry):
                grp = q // 8
                roff = (q % 8) * 128
                gv = jnp.full((_L,), grp, dtype=jnp.int32)
                g = plsc.load_gather(ins[b], [gv, iv + roff])
                ous[b][grp, pl.ds(dst0 + roff, _L)] = g
                return carry

            lax.fori_loop(0, _GCH * 8, body, 0, unroll=8)

    g_copy(0, 0).start()

    @pl.loop(0, _NCHUNK, step=2)
    def _(i):
        for b in range(2):
            v = i + b

            @pl.when(v < _NCHUNK)
            def _chunk():
                g_copy(v, b).wait()

                @pl.when(v + 1 < _NCHUNK)
                def _prefetch():
                    g_copy(v + 1, 1 - b).start()

                @pl.when(v >= 2)
                def _drain_prev():
                    s_copy(v - 2, b).wait()
                permute(b)
                s_copy(v, b).start()

    s_copy(_NCHUNK - 2, (_NCHUNK - 2) % 2).wait()
    s_copy(_NCHUNK - 1, (_NCHUNK - 1) % 2).wait()


def kernel(input):
    xp = jnp.transpose(input, (0, 2, 3, 1))
    xp = xp.reshape(_GROUPS, 8, _CT, 128)
    xp = jnp.transpose(xp, (0, 2, 1, 3)).reshape(_GROUPS, _GROW)
    out = _shuffle_groups(xp, _gather_base())
    out = out.reshape(_GROUPS, _CT, 8, 128)
    out = jnp.transpose(out, (0, 2, 1, 3)).reshape(_N, _H, _W, _C)
    return jnp.transpose(out, (0, 3, 1, 2))


# speedup vs baseline: 67.4667x; 2.2607x over previous
import functools

import jax
import jax.numpy as jnp
from jax import lax
from jax.experimental import pallas as pl
from jax.experimental.pallas import tpu as pltpu
from jax.experimental.pallas import tpu_sc as plsc

_N, _C, _H, _W = 32, 384, 56, 56
_L = 16
_CT = _C // 128
_GROUPS = _N * _H * (_W // 8)
_GROW = _CT * 8 * 128


def _gather_base():
    perm = jax.random.permutation(jax.random.key(42), _C).astype(jnp.int32)
    return (perm // 128) * 1024 + perm % 128


_NC, _NS = 2, 16
_NW = _NC * _NS
_GPW = _GROUPS // _NW
_GCH = 8
_NCHUNK = _GPW // _GCH
_CHW = _GCH * _GROW
_KB = _C // _L

_mesh = plsc.VectorSubcoreMesh(core_axis_name="c", subcore_axis_name="s")


@functools.partial(
    pl.kernel,
    mesh=_mesh,
    out_type=jax.ShapeDtypeStruct((_GROUPS, _GROW), jnp.float32),
    compiler_params=pltpu.CompilerParams(
        use_tc_tiling_on_sc=False, needs_layout_passes=False),
    scratch_types=[
        pltpu.VMEM((_C,), jnp.int32),
        pltpu.VMEM((_GCH, _GROW), jnp.float32),
        pltpu.VMEM((_GCH, _GROW), jnp.float32),
        pltpu.VMEM((_GCH, _GROW), jnp.float32),
        pltpu.VMEM((_GCH, _GROW), jnp.float32),
        pltpu.SemaphoreType.DMA,
        pltpu.SemaphoreType.DMA,
        pltpu.SemaphoreType.DMA,
        pltpu.SemaphoreType.DMA,
    ],
)
def _shuffle_groups(x_hbm, idx_hbm, out_hbm,
                    idx_v, in0, in1, ou0, ou1, g0, g1, s0, s1):
    ins, ous, gsems, ssems = (in0, in1), (ou0, ou1), (g0, g1), (s0, s1)
    wid = lax.axis_index("s") * _NC + lax.axis_index("c")
    base = wid * _GPW
    pltpu.sync_copy(idx_hbm, idx_v)

    def g_copy(v, b):
        return pltpu.make_async_copy(
            x_hbm.at[pl.ds(base + v * _GCH, _GCH)], ins[b], gsems[b])

    def s_copy(v, b):
        return pltpu.make_async_copy(
            ous[b], out_hbm.at[pl.ds(base + v * _GCH, _GCH)], ssems[b])

    def permute(b):
        for k in range(_KB):
            iv = idx_v[pl.ds(k * _L, _L)]
            ct, m = divmod(k, 8)
            dst0 = ct * 1024 + m * _L

            @plsc.parallel_loop(0, _GCH * 8, unroll=8)
            def _sweep(q):
                grp = q // 8
                roff = (q % 8) * 128
                gv = jnp.full((_L,), grp, dtype=jnp.int32)
                g = plsc.load_gather(ins[b], [gv, iv + roff])
                ous[b][grp, pl.ds(dst0 + roff, _L)] = g

    g_copy(0, 0).start()

    @pl.loop(0, _NCHUNK, step=2)
    def _(i):
        for b in range(2):
            v = i + b

            @pl.when(v < _NCHUNK)
            def _chunk():
                g_copy(v, b).wait()

                @pl.when(v + 1 < _NCHUNK)
                def _prefetch():
                    g_copy(v + 1, 1 - b).start()

                @pl.when(v >= 2)
                def _drain_prev():
                    s_copy(v - 2, b).wait()
                permute(b)
                s_copy(v, b).start()

    s_copy(_NCHUNK - 2, (_NCHUNK - 2) % 2).wait()
    s_copy(_NCHUNK - 1, (_NCHUNK - 1) % 2).wait()


def kernel(input):
    xp = jnp.transpose(input, (0, 2, 3, 1))
    xp = xp.reshape(_GROUPS, 8, _CT, 128)
    xp = jnp.transpose(xp, (0, 2, 1, 3)).reshape(_GROUPS, _GROW)
    out = _shuffle_groups(xp, _gather_base())
    out = out.reshape(_GROUPS, _CT, 8, 128)
    out = jnp.transpose(out, (0, 2, 1, 3)).reshape(_N, _H, _W, _C)
    return jnp.transpose(out, (0, 3, 1, 2))


# speedup vs baseline: 67.6514x; 1.0027x over previous
import functools

import jax
import jax.numpy as jnp
from jax import lax
from jax.experimental import pallas as pl
from jax.experimental.pallas import tpu as pltpu
from jax.experimental.pallas import tpu_sc as plsc

_N, _C, _H, _W = 32, 384, 56, 56
_L = 16
_CT = _C // 128
_GROUPS = _N * _H * (_W // 8)
_GROW = _CT * 8 * 128


def _gather_base():
    perm = jax.random.permutation(jax.random.key(42), _C).astype(jnp.int32)
    return (perm // 128) * 1024 + perm % 128


_NC, _NS = 2, 16
_NW = _NC * _NS
_GPW = _GROUPS // _NW
_GCH = 8
_NCHUNK = _GPW // _GCH
_CHW = _GCH * _GROW
_KB = _C // _L

_mesh = plsc.VectorSubcoreMesh(core_axis_name="c", subcore_axis_name="s")


@functools.partial(
    pl.kernel,
    mesh=_mesh,
    out_type=jax.ShapeDtypeStruct((_GROUPS * _GROW,), jnp.float32),
    compiler_params=pltpu.CompilerParams(
        use_tc_tiling_on_sc=False, needs_layout_passes=False),
    scratch_types=[
        pltpu.VMEM((_C,), jnp.int32),
        pltpu.VMEM((_CHW,), jnp.float32),
        pltpu.VMEM((_CHW,), jnp.float32),
        pltpu.VMEM((_CHW,), jnp.float32),
        pltpu.VMEM((_CHW,), jnp.float32),
        pltpu.SemaphoreType.DMA,
        pltpu.SemaphoreType.DMA,
        pltpu.SemaphoreType.DMA,
        pltpu.SemaphoreType.DMA,
    ],
)
def _shuffle_groups(x_hbm, idx_hbm, out_hbm,
                    idx_v, in0, in1, ou0, ou1, g0, g1, s0, s1):
    ins, ous, gsems, ssems = (in0, in1), (ou0, ou1), (g0, g1), (s0, s1)
    wid = lax.axis_index("s") * _NC + lax.axis_index("c")
    base = wid * _GPW * _GROW
    pltpu.sync_copy(idx_hbm, idx_v)

    def g_copy(v, b):
        return pltpu.make_async_copy(
            x_hbm.at[pl.ds(base + v * _CHW, _CHW)], ins[b], gsems[b])

    def s_copy(v, b):
        return pltpu.make_async_copy(
            ous[b], out_hbm.at[pl.ds(base + v * _CHW, _CHW)], ssems[b])

    def permute(b):
        for k in range(_KB):
            iv = idx_v[pl.ds(k * _L, _L)]
            ct, m = divmod(k, 8)
            dst0 = ct * 1024 + m * _L

            @plsc.parallel_loop(0, _GCH * 8, unroll=8)
            def _sweep(q):
                off = (q // 8) * _GROW + (q % 8) * 128
                g = plsc.load_gather(ins[b], [iv + off])
                ous[b][pl.ds(dst0 + off, _L)] = g

    g_copy(0, 0).start()

    @pl.loop(0, _NCHUNK, step=2)
    def _(i):
        for b in range(2):
            v = i + b

            @pl.when(v < _NCHUNK)
            def _chunk():
                g_copy(v, b).wait()

                @pl.when(v + 1 < _NCHUNK)
                def _prefetch():
                    g_copy(v + 1, 1 - b).start()

                @pl.when(v >= 2)
                def _drain_prev():
                    s_copy(v - 2, b).wait()
                permute(b)
                s_copy(v, b).start()

    s_copy(_NCHUNK - 2, (_NCHUNK - 2) % 2).wait()
    s_copy(_NCHUNK - 1, (_NCHUNK - 1) % 2).wait()


def kernel(input):
    xp = jnp.transpose(input, (0, 2, 3, 1))
    xp = xp.reshape(_GROUPS, 8, _CT, 128)
    xp = jnp.transpose(xp, (0, 2, 1, 3)).reshape(_GROUPS * _GROW)
    out = _shuffle_groups(xp, _gather_base())
    out = out.reshape(_GROUPS, _CT, 8, 128)
    out = jnp.transpose(out, (0, 2, 1, 3)).reshape(_N, _H, _W, _C)
    return jnp.transpose(out, (0, 3, 1, 2))


# speedup vs baseline: 79.2008x; 1.1707x over previous
import functools

import jax
import jax.numpy as jnp
from jax import lax
from jax.experimental import pallas as pl
from jax.experimental.pallas import tpu as pltpu
from jax.experimental.pallas import tpu_sc as plsc

_N, _C, _H, _W = 32, 384, 56, 56
_L = 16
_CT = _C // 128
_GROUPS = _N * _H * (_W // 8)
_GROW = _CT * 8 * 128


def _gather_base():
    perm = jax.random.permutation(jax.random.key(42), _C).astype(jnp.int32)
    return (perm // 128) * 1024 + perm % 128


_NC, _NS = 2, 16
_NW = _NC * _NS
_GPW = _GROUPS // _NW
_GCH = 8
_NCHUNK = _GPW // _GCH
_CHW = _GCH * _GROW
_KB = _C // _L

_mesh = plsc.VectorSubcoreMesh(core_axis_name="c", subcore_axis_name="s")


@functools.partial(
    pl.kernel,
    mesh=_mesh,
    out_type=jax.ShapeDtypeStruct((_GROUPS * _GROW,), jnp.float32),
    compiler_params=pltpu.CompilerParams(
        use_tc_tiling_on_sc=False, needs_layout_passes=False),
    scratch_types=[
        pltpu.VMEM((_C,), jnp.int32),
        pltpu.VMEM((_CHW,), jnp.float32),
        pltpu.VMEM((_CHW,), jnp.float32),
        pltpu.VMEM((_CHW,), jnp.float32),
        pltpu.VMEM((_CHW,), jnp.float32),
        pltpu.SemaphoreType.DMA,
        pltpu.SemaphoreType.DMA,
        pltpu.SemaphoreType.DMA,
        pltpu.SemaphoreType.DMA,
    ],
)
def _shuffle_groups(x_hbm, idx_hbm, out_hbm,
                    idx_v, in0, in1, ou0, ou1, g0, g1, s0, s1):
    ins, ous, gsems, ssems = (in0, in1), (ou0, ou1), (g0, g1), (s0, s1)
    wid = lax.axis_index("s") * _NC + lax.axis_index("c")
    base = wid * _GPW * _GROW
    pltpu.sync_copy(idx_hbm, idx_v)

    def g_copy(v, b):
        return pltpu.make_async_copy(
            x_hbm.at[pl.ds(base + v * _CHW, _CHW)], ins[b], gsems[b])

    def s_copy(v, b):
        return pltpu.make_async_copy(
            ous[b], out_hbm.at[pl.ds(base + v * _CHW, _CHW)], ssems[b])

    ivs = [idx_v[pl.ds(k * _L, _L)] for k in range(_KB)]

    def permute(b):
        @plsc.parallel_loop(0, _GCH * 8, unroll=2)
        def _sweep(q):
            off = (q // 8) * _GROW + (q % 8) * 128
            for k in range(_KB):
                ct, m = divmod(k, 8)
                dst0 = ct * 1024 + m * _L
                g = plsc.load_gather(ins[b], [ivs[k] + off])
                ous[b][pl.ds(dst0 + off, _L)] = g

    g_copy(0, 0).start()

    @pl.loop(0, _NCHUNK, step=2)
    def _(i):
        for b in range(2):
            v = i + b

            @pl.when(v < _NCHUNK)
            def _chunk():
                g_copy(v, b).wait()

                @pl.when(v + 1 < _NCHUNK)
                def _prefetch():
                    g_copy(v + 1, 1 - b).start()

                @pl.when(v >= 2)
                def _drain_prev():
                    s_copy(v - 2, b).wait()
                permute(b)
                s_copy(v, b).start()

    s_copy(_NCHUNK - 2, (_NCHUNK - 2) % 2).wait()
    s_copy(_NCHUNK - 1, (_NCHUNK - 1) % 2).wait()


def kernel(input):
    xp = jnp.transpose(input, (0, 2, 3, 1))
    xp = xp.reshape(_GROUPS, 8, _CT, 128)
    xp = jnp.transpose(xp, (0, 2, 1, 3)).reshape(_GROUPS * _GROW)
    out = _shuffle_groups(xp, _gather_base())
    out = out.reshape(_GROUPS, _CT, 8, 128)
    out = jnp.transpose(out, (0, 2, 1, 3)).reshape(_N, _H, _W, _C)
    return jnp.transpose(out, (0, 3, 1, 2))


# speedup vs baseline: 80.6580x; 1.0184x over previous
import functools

import jax
import jax.numpy as jnp
from jax import lax
from jax.experimental import pallas as pl
from jax.experimental.pallas import tpu as pltpu
from jax.experimental.pallas import tpu_sc as plsc

_N, _C, _H, _W = 32, 384, 56, 56
_L = 16
_CT = _C // 128
_GROUPS = _N * _H * (_W // 8)
_GROW = _CT * 8 * 128


def _gather_base():
    perm = jax.random.permutation(jax.random.key(42), _C).astype(jnp.int32)
    return (perm // 128) * 1024 + perm % 128


_NC, _NS = 2, 16
_NW = _NC * _NS
_GPW = _GROUPS // _NW
_GCH = 8
_NCHUNK = _GPW // _GCH
_CHW = _GCH * _GROW
_KB = _C // _L

_mesh = plsc.VectorSubcoreMesh(core_axis_name="c", subcore_axis_name="s")


@functools.partial(
    pl.kernel,
    mesh=_mesh,
    out_type=jax.ShapeDtypeStruct((_GROUPS * _GROW,), jnp.float32),
    compiler_params=pltpu.CompilerParams(
        use_tc_tiling_on_sc=False, needs_layout_passes=False),
    scratch_types=[
        pltpu.VMEM((_C,), jnp.int32),
        pltpu.VMEM((_CHW,), jnp.float32),
        pltpu.VMEM((_CHW,), jnp.float32),
        pltpu.VMEM((_CHW,), jnp.float32),
        pltpu.VMEM((_CHW,), jnp.float32),
        pltpu.SemaphoreType.DMA,
        pltpu.SemaphoreType.DMA,
        pltpu.SemaphoreType.DMA,
        pltpu.SemaphoreType.DMA,
    ],
)
def _shuffle_groups(x_hbm, idx_hbm, out_hbm,
                    idx_v, in0, in1, ou0, ou1, g0, g1, s0, s1):
    ins, ous, gsems, ssems = (in0, in1), (ou0, ou1), (g0, g1), (s0, s1)
    wid = lax.axis_index("s") * _NC + lax.axis_index("c")
    base = wid * _GPW * _GROW
    pltpu.sync_copy(idx_hbm, idx_v)

    def g_copy(v, b):
        return pltpu.make_async_copy(
            x_hbm.at[pl.ds(base + v * _CHW, _CHW)], ins[b], gsems[b])

    def s_copy(v, b):
        return pltpu.make_async_copy(
            ous[b], out_hbm.at[pl.ds(base + v * _CHW, _CHW)], ssems[b])

    ivs = [idx_v[pl.ds(k * _L, _L)] for k in range(_KB)]

    def permute(b):
        @plsc.parallel_loop(0, _GCH * 8, unroll=4)
        def _sweep(q):
            off = (q // 8) * _GROW + (q % 8) * 128
            for k in range(_KB):
                ct, m = divmod(k, 8)
                dst0 = ct * 1024 + m * _L
                g = plsc.load_gather(ins[b], [ivs[k] + off])
                ous[b][pl.ds(dst0 + off, _L)] = g

    g_copy(0, 0).start()

    @pl.loop(0, _NCHUNK, step=2)
    def _(i):
        for b in range(2):
            v = i + b

            @pl.when(v < _NCHUNK)
            def _chunk():
                g_copy(v, b).wait()

                @pl.when(v + 1 < _NCHUNK)
                def _prefetch():
                    g_copy(v + 1, 1 - b).start()

                @pl.when(v >= 2)
                def _drain_prev():
                    s_copy(v - 2, b).wait()
                permute(b)
                s_copy(v, b).start()

    s_copy(_NCHUNK - 2, (_NCHUNK - 2) % 2).wait()
    s_copy(_NCHUNK - 1, (_NCHUNK - 1) % 2).wait()


def kernel(input):
    xp = jnp.transpose(input, (0, 2, 3, 1))
    xp = xp.reshape(_GROUPS, 8, _CT, 128)
    xp = jnp.transpose(xp, (0, 2, 1, 3)).reshape(_GROUPS * _GROW)
    out = _shuffle_groups(xp, _gather_base())
    out = out.reshape(_GROUPS, _CT, 8, 128)
    out = jnp.transpose(out, (0, 2, 1, 3)).reshape(_N, _H, _W, _C)
    return jnp.transpose(out, (0, 3, 1, 2))
